# baseline (speedup 1.0000x reference)
import jax
import jax.numpy as jnp
from jax import lax
from jax.experimental import pallas as pl
from jax.experimental.pallas import tpu as pltpu
from jax.experimental.pallas import tpu_sc as plsc

VOCAB = 100000
E = 128
S = 20
M = 50
B = 1024

NSEG = B * M
RPS = S
CH = 4
GIDX = CH * RPS
GPAD = 80
NCHUNKS = NSEG // CH
NWORKERS = 32
CPW = NCHUNKS // NWORKERS
SPW = NSEG // NWORKERS
NBUF = 5
CR = 125
RPT = VOCAB // 16

EB = E // 16

A_COEF = [(s - 9.5) / 640.0 for s in range(S - 1)]


def _sc_body(idx_hbm, table_hbm, te_hbm, out_hbm, packed_hbm,
             idx_v, te_v, ci0, ci1, co0, co1, csem0, csem1, *bufs):
    rbufs = bufs[0:NBUF]
    obufs = bufs[NBUF:2 * NBUF]
    gsems = bufs[2 * NBUF:3 * NBUF]
    osems = bufs[3 * NBUF:4 * NBUF]

    wid = lax.axis_index("s") * 2 + lax.axis_index("c")
    chunk0 = wid * CPW
    seg0 = wid * SPW

    sub = lax.axis_index("s")
    row00 = sub * RPT
    cis = (ci0, ci1)
    cos = (co0, co1)
    csems = (csem0, csem1)

    def start_cin(i, buf, sem):
        pltpu.make_async_copy(
            table_hbm.at[pl.ds(row00 + i * CR, CR)], buf, sem).start()

    start_cin(0, ci0, csem0)
    start_cin(1, ci1, csem1)

    def cvt_body(g, carry):
        for b in range(2):
            i = 2 * g + b
            ci, co, csem = cis[b], cos[b], csems[b]
            pltpu.make_async_copy(
                table_hbm.at[pl.ds(0, CR)], ci, csem).wait()
            for r in range(CR):
                for c in range(4):
                    a = plsc.bitcast(ci[r, pl.ds(c * 16, 16)], jnp.int32)
                    d = plsc.bitcast(ci[r, pl.ds(64 + c * 16, 16)], jnp.int32)
                    ar = a + 0x7FFF + ((a >> 16) & 1)
                    dr = d + 0x7FFF + ((d >> 16) & 1)
                    co[r, pl.ds(c * 16, 16)] = (
                        ((ar >> 16) & 0xFFFF) | (dr & jnp.int32(-65536)))
            pltpu.sync_copy(co, packed_hbm.at[pl.ds(row00 + i * CR, CR)])

            @pl.when(i + 2 < RPT // CR)
            def _():
                start_cin(i + 2, ci, csem)
        return carry

    lax.fori_loop(0, RPT // CR // 2, cvt_body, 0)
    plsc.subcore_barrier()

    pltpu.sync_copy(idx_hbm.at[pl.ds(chunk0, CPW)], idx_v)
    pltpu.sync_copy(te_hbm, te_v)

    lane = lax.iota(jnp.int32, 16).astype(jnp.float32)
    bvecs = [lane + (eb * 16 - 63.5) for eb in range(EB)]

    def start_gather(it, buf, sem):
        pltpu.make_async_copy(packed_hbm.at[idx_v.at[it]], buf, sem).start()

    def wait_gather(buf, sem):
        pltpu.make_async_copy(packed_hbm.at[idx_v.at[0]], buf, sem).wait()

    for b in range(NBUF):
        start_gather(b, rbufs[b], gsems[b])

    def body(g, carry):
        for b in range(NBUF):
            it = NBUF * g + b
            rb, ob, gsem, osem = rbufs[b], obufs[b], gsems[b], osems[b]
            wait_gather(rb, gsem)

            @pl.when(it >= NBUF)
            def _():
                pltpu.make_async_copy(
                    ob, out_hbm.at[pl.ds(0, CH)], osem).wait()

            seg_in_batch = it * CH

            for j in range(CH):
                r0 = j * RPS
                mj = lax.rem(seg_in_batch + j, M)
                for c in range(4):
                    sl = pl.ds(c * 16, 16)

                    def rowpair(r):
                        w = rb[r, sl]
                        lo = plsc.bitcast(w << 16, jnp.float32)
                        hi = plsc.bitcast(w & jnp.int32(-65536), jnp.float32)
                        return lo, hi

                    lo, hi = rowpair(r0)
                    acc_lo = A_COEF[0] * lo
                    acc_hi = A_COEF[0] * hi
                    for s in range(1, S - 1):
                        lo, hi = rowpair(r0 + s)
                        acc_lo = acc_lo + A_COEF[s] * lo
                        acc_hi = acc_hi + A_COEF[s] * hi
                    l19, h19 = rowpair(r0 + S - 1)
                    tlo = te_v[pl.ds(mj * E + c * 16, 16)]
                    thi = te_v[pl.ds(mj * E + 64 + c * 16, 16)]
                    ob[j, pl.ds(c * 16, 16)] = (
                        acc_lo * bvecs[c] + l19 + tlo)
                    ob[j, pl.ds(64 + c * 16, 16)] = (
                        acc_hi * bvecs[c + 4] + h19 + thi)

            pltpu.make_async_copy(
                ob, out_hbm.at[pl.ds(seg0 + it * CH, CH)], osem).start()

            @pl.when(it + NBUF < CPW)
            def _():
                start_gather(it + NBUF, rb, gsem)
        return carry

    lax.fori_loop(0, CPW // NBUF, body, 0)

    for b in range(NBUF):
        pltpu.make_async_copy(
            obufs[b], out_hbm.at[pl.ds(0, CH)], osems[b]).wait()


@jax.jit
def kernel(x, embedding_table, temporal_table):
    idx = x.astype(jnp.int32).reshape(NCHUNKS, GIDX)

    table = embedding_table
    te = temporal_table.reshape(M * E)

    mesh = plsc.VectorSubcoreMesh(core_axis_name="c", subcore_axis_name="s")
    run = pl.kernel(
        _sc_body,
        mesh=mesh,
        compiler_params=pltpu.CompilerParams(
            needs_layout_passes=False, use_tc_tiling_on_sc=False),
        out_type=[jax.ShapeDtypeStruct((NSEG, E), jnp.float32),
                  jax.ShapeDtypeStruct((VOCAB, E // 2), jnp.int32)],
        scratch_types=(
            [pltpu.VMEM((CPW, GPAD), jnp.int32),
             pltpu.VMEM((M * E,), jnp.float32),
             pltpu.VMEM((CR, E), jnp.float32),
             pltpu.VMEM((CR, E), jnp.float32),
             pltpu.VMEM((CR, E // 2), jnp.int32),
             pltpu.VMEM((CR, E // 2), jnp.int32),
             pltpu.SemaphoreType.DMA,
             pltpu.SemaphoreType.DMA]
            + [pltpu.VMEM((GPAD, E // 2), jnp.int32) for _ in range(NBUF)]
            + [pltpu.VMEM((CH, E), jnp.float32) for _ in range(NBUF)]
            + [pltpu.SemaphoreType.DMA for _ in range(2 * NBUF)]
        ),
    )
    out, _ = run(idx, table, te)
    return out.reshape(B, M, E)


# speedup vs baseline: 1.0262x; 1.0262x over previous
import jax
import jax.numpy as jnp
from jax import lax
from jax.experimental import pallas as pl
from jax.experimental.pallas import tpu as pltpu
from jax.experimental.pallas import tpu_sc as plsc

VOCAB = 100000
E = 128
S = 20
M = 50
B = 1024

NSEG = B * M
RPS = S
CH = 4
GIDX = CH * RPS
GPAD = 80
NCHUNKS = NSEG // CH
NWORKERS = 32
CPW = NCHUNKS // NWORKERS
SPW = NSEG // NWORKERS
NBUF = 4
CR = 125
RPT = VOCAB // 16

EB = E // 16

A_COEF = [(s - 9.5) / 640.0 for s in range(S - 1)]


def _sc_body(idx_hbm, table_hbm, te_hbm, out_hbm, packed_hbm,
             idx_v, te_v, ci0, ci1, co0, co1, csem0, csem1, *bufs):
    rbufs = bufs[0:NBUF]
    obufs = bufs[NBUF:2 * NBUF]
    gsems = bufs[2 * NBUF:3 * NBUF]
    osems = bufs[3 * NBUF:4 * NBUF]

    wid = lax.axis_index("s") * 2 + lax.axis_index("c")
    chunk0 = wid * CPW
    seg0 = wid * SPW

    sub = lax.axis_index("s")
    row00 = sub * RPT
    cis = (ci0, ci1)
    cos = (co0, co1)
    csems = (csem0, csem1)

    def start_cin(i, buf, sem):
        pltpu.make_async_copy(
            table_hbm.at[pl.ds(row00 + i * CR, CR)], buf, sem).start()

    start_cin(0, ci0, csem0)
    start_cin(1, ci1, csem1)

    def cvt_body(g, carry):
        for b in range(2):
            i = 2 * g + b
            ci, co, csem = cis[b], cos[b], csems[b]
            pltpu.make_async_copy(
                table_hbm.at[pl.ds(0, CR)], ci, csem).wait()
            for r in range(CR):
                for c in range(4):
                    a = plsc.bitcast(ci[r, pl.ds(c * 16, 16)], jnp.int32)
                    d = plsc.bitcast(ci[r, pl.ds(64 + c * 16, 16)], jnp.int32)
                    ar = a + 0x7FFF + ((a >> 16) & 1)
                    dr = d + 0x7FFF + ((d >> 16) & 1)
                    co[r, pl.ds(c * 16, 16)] = (
                        ((ar >> 16) & 0xFFFF) | (dr & jnp.int32(-65536)))
            pltpu.sync_copy(co, packed_hbm.at[pl.ds(row00 + i * CR, CR)])

            @pl.when(i + 2 < RPT // CR)
            def _():
                start_cin(i + 2, ci, csem)
        return carry

    lax.fori_loop(0, RPT // CR // 2, cvt_body, 0)
    plsc.subcore_barrier()

    pltpu.sync_copy(idx_hbm.at[pl.ds(chunk0, CPW)], idx_v)
    pltpu.sync_copy(te_hbm, te_v)

    lane = lax.iota(jnp.int32, 16).astype(jnp.float32)
    bvecs = [lane + (eb * 16 - 63.5) for eb in range(EB)]

    def start_gather(it, buf, sem):
        pltpu.make_async_copy(packed_hbm.at[idx_v.at[it]], buf, sem).start()

    def wait_gather(buf, sem):
        pltpu.make_async_copy(packed_hbm.at[idx_v.at[0]], buf, sem).wait()

    for b in range(NBUF):
        start_gather(b, rbufs[b], gsems[b])

    def body(g, carry):
        for b in range(NBUF):
            it = NBUF * g + b
            rb, ob, gsem, osem = rbufs[b], obufs[b], gsems[b], osems[b]
            wait_gather(rb, gsem)

            @pl.when(it >= NBUF)
            def _():
                pltpu.make_async_copy(
                    ob, out_hbm.at[pl.ds(0, CH)], osem).wait()

            seg_in_batch = it * CH

            for j in range(CH):
                r0 = j * RPS
                mj = lax.rem(seg_in_batch + j, M)
                for c in range(4):
                    sl = pl.ds(c * 16, 16)

                    def rowpair(r):
                        w = rb[r, sl]
                        lo = plsc.bitcast(w << 16, jnp.float32)
                        hi = plsc.bitcast(w & jnp.int32(-65536), jnp.float32)
                        return lo, hi

                    lo, hi = rowpair(r0)
                    acc_lo = A_COEF[0] * lo
                    acc_hi = A_COEF[0] * hi
                    for s in range(1, S - 1):
                        lo, hi = rowpair(r0 + s)
                        acc_lo = acc_lo + A_COEF[s] * lo
                        acc_hi = acc_hi + A_COEF[s] * hi
                    l19, h19 = rowpair(r0 + S - 1)
                    tlo = te_v[pl.ds(mj * E + c * 16, 16)]
                    thi = te_v[pl.ds(mj * E + 64 + c * 16, 16)]
                    ob[j, pl.ds(c * 16, 16)] = (
                        acc_lo * bvecs[c] + l19 + tlo)
                    ob[j, pl.ds(64 + c * 16, 16)] = (
                        acc_hi * bvecs[c + 4] + h19 + thi)

            pltpu.make_async_copy(
                ob, out_hbm.at[pl.ds(seg0 + it * CH, CH)], osem).start()

            @pl.when(it + NBUF < CPW)
            def _():
                start_gather(it + NBUF, rb, gsem)
        return carry

    lax.fori_loop(0, CPW // NBUF, body, 0)

    for b in range(NBUF):
        pltpu.make_async_copy(
            obufs[b], out_hbm.at[pl.ds(0, CH)], osems[b]).wait()


@jax.jit
def kernel(x, embedding_table, temporal_table):
    idx = x.astype(jnp.int32).reshape(NCHUNKS, GIDX)

    table = embedding_table
    te = temporal_table.reshape(M * E)

    mesh = plsc.VectorSubcoreMesh(core_axis_name="c", subcore_axis_name="s")
    run = pl.kernel(
        _sc_body,
        mesh=mesh,
        compiler_params=pltpu.CompilerParams(
            needs_layout_passes=False, use_tc_tiling_on_sc=False),
        out_type=[jax.ShapeDtypeStruct((NSEG, E), jnp.float32),
                  jax.ShapeDtypeStruct((VOCAB, E // 2), jnp.int32)],
        scratch_types=(
            [pltpu.VMEM((CPW, GPAD), jnp.int32),
             pltpu.VMEM((M * E,), jnp.float32),
             pltpu.VMEM((CR, E), jnp.float32),
             pltpu.VMEM((CR, E), jnp.float32),
             pltpu.VMEM((CR, E // 2), jnp.int32),
             pltpu.VMEM((CR, E // 2), jnp.int32),
             pltpu.SemaphoreType.DMA,
             pltpu.SemaphoreType.DMA]
            + [pltpu.VMEM((GPAD, E // 2), jnp.int32) for _ in range(NBUF)]
            + [pltpu.VMEM((CH, E), jnp.float32) for _ in range(NBUF)]
            + [pltpu.SemaphoreType.DMA for _ in range(2 * NBUF)]
        ),
    )
    out, _ = run(idx, table, te)
    return out.reshape(B, M, E)
